# initial kernel scaffold (unmeasured)
import jax
import jax.numpy as jnp
from jax import lax
from jax.experimental import pallas as pl
from jax.experimental.pallas import tpu as pltpu


def kernel(
    x,
):
    def body(*refs):
        pass

    out_shape = jax.ShapeDtypeStruct(..., jnp.float32)
    return pl.pallas_call(body, out_shape=out_shape)(...)


# baseline (device time: 971578 ns/iter reference)
import numpy as np

import jax
import jax.numpy as jnp
from jax import lax
from jax.experimental import pallas as pl
from jax.experimental.pallas import tpu as pltpu

N_DEV = 32

_RBLK = 256


def _cmpex_phase(ref, k, j, base0, n_rows):
    if j >= 8:
        c_rows = min(j, _RBLK)
        n_sub = j // c_rows

        def body(t, carry):
            b = t // n_sub
            s = t % n_sub
            base = base0 + b * (2 * j)
            off = base + s * c_rows
            a = ref[pl.ds(off, c_rows), :]
            c = ref[pl.ds(off + j, c_rows), :]
            mn = jnp.minimum(a, c)
            mx = jnp.maximum(a, c)
            asc = (base & k) == 0
            ref[pl.ds(off, c_rows), :] = jnp.where(asc, mn, mx)
            ref[pl.ds(off + j, c_rows), :] = jnp.where(asc, mx, mn)
            return carry

        lax.fori_loop(0, (n_rows // (2 * j)) * n_sub, body, 0)
    else:
        g = _RBLK // (2 * j)

        def body(t, carry):
            base = base0 + t * _RBLK
            x = ref[pl.ds(base, _RBLK), :]
            y = x.reshape(g, 2, j, x.shape[-1])
            a = y[:, 0]
            c = y[:, 1]
            mn = jnp.minimum(a, c)
            mx = jnp.maximum(a, c)
            if k < _RBLK:
                iota = lax.broadcasted_iota(jnp.int32, (g, 1, 1), 0)
                asc = ((iota * (2 * j)) & k) == 0
            else:
                asc = (base & k) == 0
            first = jnp.where(asc, mn, mx)
            second = jnp.where(asc, mx, mn)
            z = jnp.stack([first, second], axis=1)
            ref[pl.ds(base, _RBLK), :] = z.reshape(_RBLK, x.shape[-1])
            return carry

        lax.fori_loop(0, n_rows // _RBLK, body, 0)


def kernel(x):
    m, n = x.shape
    n_total = N_DEV * m

    def body(x_ref, out_ref, gather_ref, send_sems, recv_sems):
        p = lax.axis_index("i")
        left = lax.rem(p + N_DEV - 1, N_DEV)
        right = lax.rem(p + 1, N_DEV)

        barrier = pltpu.get_barrier_semaphore()
        for nbr in (left, right):
            pl.semaphore_signal(
                barrier,
                inc=1,
                device_id=(nbr,),
                device_id_type=pl.DeviceIdType.MESH,
            )
        pl.semaphore_wait(barrier, 2)

        my_base = p * m
        gather_ref[pl.ds(my_base, m), :] = x_ref[:, :]

        k = 2
        while k <= m:
            j = k // 2
            while j >= 1:
                _cmpex_phase(gather_ref, k, j, my_base, m)
                j //= 2
            k *= 2

        for h in range(N_DEV - 1):
            origin = lax.rem(p - h + 2 * N_DEV, N_DEV)
            rdma = pltpu.make_async_remote_copy(
                src_ref=gather_ref.at[pl.ds(origin * m, m)],
                dst_ref=gather_ref.at[pl.ds(origin * m, m)],
                send_sem=send_sems.at[h],
                recv_sem=recv_sems.at[h],
                device_id=(right,),
                device_id_type=pl.DeviceIdType.MESH,
            )
            rdma.start()
            rdma.wait()

        k = 2 * m
        while k <= n_total:
            j = k // 2
            while j >= 1:
                _cmpex_phase(gather_ref, k, j, 0, n_total)
                j //= 2
            k *= 2

        out_ref[:, :] = gather_ref[pl.ds(my_base, m), :]

    return pl.pallas_call(
        body,
        out_shape=jax.ShapeDtypeStruct((m, n), x.dtype),
        in_specs=[pl.BlockSpec(memory_space=pltpu.VMEM)],
        out_specs=pl.BlockSpec(memory_space=pltpu.VMEM),
        scratch_shapes=[
            pltpu.VMEM((n_total, n), x.dtype),
            pltpu.SemaphoreType.DMA((N_DEV - 1,)),
            pltpu.SemaphoreType.DMA((N_DEV - 1,)),
        ],
        compiler_params=pltpu.CompilerParams(
            collective_id=0,
            vmem_limit_bytes=100 * 1024 * 1024,
        ),
    )(x)


# device time: 286981 ns/iter; 3.3855x vs baseline; 3.3855x over previous
import jax
import jax.numpy as jnp
from jax import lax
from jax.experimental import pallas as pl
from jax.experimental.pallas import tpu as pltpu

N_DEV = 32

_RBLK = 256


def _cmpex_phase(ref, k, j, n_rows, gbase):
    if j >= 8:
        c_rows = min(j, _RBLK)
        n_sub = j // c_rows

        def body(t, carry):
            b = t // n_sub
            s = t % n_sub
            base = b * (2 * j)
            off = base + s * c_rows
            a = ref[pl.ds(off, c_rows), :]
            c = ref[pl.ds(off + j, c_rows), :]
            mn = jnp.minimum(a, c)
            mx = jnp.maximum(a, c)
            asc = ((gbase + base) & k) == 0
            ref[pl.ds(off, c_rows), :] = jnp.where(asc, mn, mx)
            ref[pl.ds(off + j, c_rows), :] = jnp.where(asc, mx, mn)
            return carry

        lax.fori_loop(0, (n_rows // (2 * j)) * n_sub, body, 0)
    else:
        g = _RBLK // (2 * j)

        def body(t, carry):
            base = t * _RBLK
            x = ref[pl.ds(base, _RBLK), :]
            y = x.reshape(g, 2, j, x.shape[-1])
            a = y[:, 0]
            c = y[:, 1]
            mn = jnp.minimum(a, c)
            mx = jnp.maximum(a, c)
            if k < _RBLK:
                iota = lax.broadcasted_iota(jnp.int32, (g, 1, 1), 0)
                asc = ((iota * (2 * j)) & k) == 0
            else:
                asc = ((gbase + base) & k) == 0
            first = jnp.where(asc, mn, mx)
            second = jnp.where(asc, mx, mn)
            z = jnp.stack([first, second], axis=1)
            ref[pl.ds(base, _RBLK), :] = z.reshape(_RBLK, x.shape[-1])
            return carry

        lax.fori_loop(0, n_rows // _RBLK, body, 0)


def kernel(x):
    m, n = x.shape
    n_exch = 15

    def body(x_ref, out_ref, staging_ref, send_sems, recv_sems):
        p = lax.axis_index("i")
        gbase = p * m

        partners = [1, 2, 4, 8, 16]
        barrier = pltpu.get_barrier_semaphore()
        for d in partners:
            pl.semaphore_signal(
                barrier,
                inc=1,
                device_id=(jnp.bitwise_xor(p, d),),
                device_id_type=pl.DeviceIdType.MESH,
            )
        pl.semaphore_wait(barrier, len(partners))

        out_ref[:, :] = x_ref[:, :]

        k = 2
        while k <= m:
            j = k // 2
            while j >= 1:
                _cmpex_phase(out_ref, k, j, m, gbase)
                j //= 2
            k *= 2

        t = 0
        K = 2
        while K <= N_DEV:
            d = K // 2
            while d >= 1:
                q = jnp.bitwise_xor(p, d)
                rdma = pltpu.make_async_remote_copy(
                    src_ref=out_ref,
                    dst_ref=staging_ref.at[t],
                    send_sem=send_sems.at[t],
                    recv_sem=recv_sems.at[t],
                    device_id=(q,),
                    device_id_type=pl.DeviceIdType.MESH,
                )
                rdma.start()
                rdma.wait()
                asc = (p & K) == 0
                lower = (p & d) == 0
                keep_min = asc == lower
                for s in range(m // _RBLK):
                    a = out_ref[pl.ds(s * _RBLK, _RBLK), :]
                    b = staging_ref[t, pl.ds(s * _RBLK, _RBLK), :]
                    mn = jnp.minimum(a, b)
                    mx = jnp.maximum(a, b)
                    out_ref[pl.ds(s * _RBLK, _RBLK), :] = jnp.where(
                        keep_min, mn, mx
                    )
                t += 1
                d //= 2
            kk = K * m
            j = m // 2
            while j >= 1:
                _cmpex_phase(out_ref, kk, j, m, gbase)
                j //= 2
            K *= 2

        def _exit(second_barrier):
            for d in partners:
                pl.semaphore_signal(
                    second_barrier,
                    inc=1,
                    device_id=(jnp.bitwise_xor(p, d),),
                    device_id_type=pl.DeviceIdType.MESH,
                )
            pl.semaphore_wait(second_barrier, len(partners))

        pl.run_scoped(_exit, second_barrier=pltpu.SemaphoreType.REGULAR)

    return pl.pallas_call(
        body,
        out_shape=jax.ShapeDtypeStruct((m, n), x.dtype),
        in_specs=[pl.BlockSpec(memory_space=pltpu.VMEM)],
        out_specs=pl.BlockSpec(memory_space=pltpu.VMEM),
        scratch_shapes=[
            pltpu.VMEM((n_exch, m, n), x.dtype),
            pltpu.SemaphoreType.DMA((n_exch,)),
            pltpu.SemaphoreType.DMA((n_exch,)),
        ],
        compiler_params=pltpu.CompilerParams(
            collective_id=0,
            vmem_limit_bytes=100 * 1024 * 1024,
        ),
    )(x)


# device time: 261855 ns/iter; 3.7104x vs baseline; 1.0960x over previous
import jax
import jax.numpy as jnp
from jax import lax
from jax.experimental import pallas as pl
from jax.experimental.pallas import tpu as pltpu

N_DEV = 32
N_GRP = 2

_RBLK = 256


def _cmpex_phase(ref, gidx, k, j, n_rows, gbase):
    if j >= 8:
        c_rows = min(j, _RBLK)
        n_sub = j // c_rows

        def body(t, carry):
            b = t // n_sub
            s = t % n_sub
            base = b * (2 * j)
            off = base + s * c_rows
            a = ref[gidx, pl.ds(off, c_rows), :]
            c = ref[gidx, pl.ds(off + j, c_rows), :]
            mn = jnp.minimum(a, c)
            mx = jnp.maximum(a, c)
            asc = ((gbase + base) & k) == 0
            ref[gidx, pl.ds(off, c_rows), :] = jnp.where(asc, mn, mx)
            ref[gidx, pl.ds(off + j, c_rows), :] = jnp.where(asc, mx, mn)
            return carry

        lax.fori_loop(0, (n_rows // (2 * j)) * n_sub, body, 0)
    else:
        g = _RBLK // (2 * j)

        def body(t, carry):
            base = t * _RBLK
            x = ref[gidx, pl.ds(base, _RBLK), :]
            y = x.reshape(g, 2, j, x.shape[-1])
            a = y[:, 0]
            c = y[:, 1]
            mn = jnp.minimum(a, c)
            mx = jnp.maximum(a, c)
            if k < _RBLK:
                iota = lax.broadcasted_iota(jnp.int32, (g, 1, 1), 0)
                asc = ((iota * (2 * j)) & k) == 0
            else:
                asc = ((gbase + base) & k) == 0
            first = jnp.where(asc, mn, mx)
            second = jnp.where(asc, mx, mn)
            z = jnp.stack([first, second], axis=1)
            ref[gidx, pl.ds(base, _RBLK), :] = z.reshape(_RBLK, x.shape[-1])
            return carry

        lax.fori_loop(0, n_rows // _RBLK, body, 0)


def kernel(x):
    m, n = x.shape
    cw = n // N_GRP
    n_exch = 15

    ops = []
    t = 0
    K = 2
    while K <= N_DEV:
        d = K // 2
        while d >= 1:
            ops.append(("ex", t, K, d))
            t += 1
            d //= 2
        ops.append(("loc", K))
        K *= 2

    def body(x_ref, out_ref, work_ref, staging_ref, send_sems, recv_sems):
        p = lax.axis_index("i")
        gbase = p * m

        partners = [1, 2, 4, 8, 16]
        barrier = pltpu.get_barrier_semaphore()
        for d in partners:
            pl.semaphore_signal(
                barrier,
                inc=1,
                device_id=(jnp.bitwise_xor(p, d),),
                device_id_type=pl.DeviceIdType.MESH,
            )
        pl.semaphore_wait(barrier, len(partners))

        def make_ex(t, d, g):
            return pltpu.make_async_remote_copy(
                src_ref=work_ref.at[g],
                dst_ref=staging_ref.at[t, g],
                send_sem=send_sems.at[t, g],
                recv_sem=recv_sems.at[t, g],
                device_id=(jnp.bitwise_xor(p, d),),
                device_id_type=pl.DeviceIdType.MESH,
            )

        def keep(t, K, d, g):
            asc = (p & K) == 0
            lower = (p & d) == 0
            keep_min = asc == lower
            for s in range(m // _RBLK):
                rows = pl.ds(s * _RBLK, _RBLK)
                a = work_ref[g, rows, :]
                b = staging_ref[t, g, rows, :]
                mn = jnp.minimum(a, b)
                mx = jnp.maximum(a, b)
                work_ref[g, rows, :] = jnp.where(keep_min, mn, mx)

        rdmas = {}

        for g in range(N_GRP):
            for s in range(m // _RBLK):
                rows = pl.ds(s * _RBLK, _RBLK)
                work_ref[g, rows, :] = x_ref[rows, pl.ds(g * cw, cw)]
            k = 2
            while k <= m:
                j = k // 2
                while j >= 1:
                    _cmpex_phase(work_ref, g, k, j, m, gbase)
                    j //= 2
                k *= 2
            rdmas[(0, g)] = make_ex(0, ops[0][3], g)
            rdmas[(0, g)].start()

        for idx, op in enumerate(ops):
            for g in range(N_GRP):
                if op[0] == "ex":
                    _, t, K, d = op
                    rdmas[(t, g)].wait()
                    keep(t, K, d, g)
                else:
                    kk = op[1] * m
                    j = m // 2
                    while j >= 1:
                        _cmpex_phase(work_ref, g, kk, j, m, gbase)
                        j //= 2
                if idx + 1 < len(ops) and ops[idx + 1][0] == "ex":
                    t2, d2 = ops[idx + 1][1], ops[idx + 1][3]
                    rdmas[(t2, g)] = make_ex(t2, d2, g)
                    rdmas[(t2, g)].start()

        for g in range(N_GRP):
            for s in range(m // _RBLK):
                rows = pl.ds(s * _RBLK, _RBLK)
                out_ref[rows, pl.ds(g * cw, cw)] = work_ref[g, rows, :]

        def _exit(second_barrier):
            for d in partners:
                pl.semaphore_signal(
                    second_barrier,
                    inc=1,
                    device_id=(jnp.bitwise_xor(p, d),),
                    device_id_type=pl.DeviceIdType.MESH,
                )
            pl.semaphore_wait(second_barrier, len(partners))

        pl.run_scoped(_exit, second_barrier=pltpu.SemaphoreType.REGULAR)

    return pl.pallas_call(
        body,
        out_shape=jax.ShapeDtypeStruct((m, n), x.dtype),
        in_specs=[pl.BlockSpec(memory_space=pltpu.VMEM)],
        out_specs=pl.BlockSpec(memory_space=pltpu.VMEM),
        scratch_shapes=[
            pltpu.VMEM((N_GRP, m, cw), x.dtype),
            pltpu.VMEM((n_exch, N_GRP, m, cw), x.dtype),
            pltpu.SemaphoreType.DMA((n_exch, N_GRP)),
            pltpu.SemaphoreType.DMA((n_exch, N_GRP)),
        ],
        compiler_params=pltpu.CompilerParams(
            collective_id=0,
            vmem_limit_bytes=100 * 1024 * 1024,
        ),
    )(x)


# device time: 255641 ns/iter; 3.8006x vs baseline; 1.0243x over previous
import jax
import jax.numpy as jnp
from jax import lax
from jax.experimental import pallas as pl
from jax.experimental.pallas import tpu as pltpu

N_DEV = 32
N_GRP = 2

_RBLK = 256


def _cmpex_phase(ref, gidx, k, j, n_rows, gbase):
    if j >= 8:
        c_rows = min(j, _RBLK)
        n_sub = j // c_rows

        def body(t, carry):
            b = t // n_sub
            s = t % n_sub
            base = b * (2 * j)
            off = base + s * c_rows
            a = ref[gidx, pl.ds(off, c_rows), :]
            c = ref[gidx, pl.ds(off + j, c_rows), :]
            mn = jnp.minimum(a, c)
            mx = jnp.maximum(a, c)
            asc = ((gbase + base) & k) == 0
            ref[gidx, pl.ds(off, c_rows), :] = jnp.where(asc, mn, mx)
            ref[gidx, pl.ds(off + j, c_rows), :] = jnp.where(asc, mx, mn)
            return carry

        lax.fori_loop(0, (n_rows // (2 * j)) * n_sub, body, 0)
    else:
        g = _RBLK // (2 * j)

        def body(t, carry):
            base = t * _RBLK
            x = ref[gidx, pl.ds(base, _RBLK), :]
            y = x.reshape(g, 2, j, x.shape[-1])
            a = y[:, 0]
            c = y[:, 1]
            mn = jnp.minimum(a, c)
            mx = jnp.maximum(a, c)
            if k < _RBLK:
                iota = lax.broadcasted_iota(jnp.int32, (g, 1, 1), 0)
                asc = ((iota * (2 * j)) & k) == 0
            else:
                asc = ((gbase + base) & k) == 0
            first = jnp.where(asc, mn, mx)
            second = jnp.where(asc, mx, mn)
            z = jnp.stack([first, second], axis=1)
            ref[gidx, pl.ds(base, _RBLK), :] = z.reshape(_RBLK, x.shape[-1])
            return carry

        lax.fori_loop(0, n_rows // _RBLK, body, 0)


def kernel(x):
    m, n = x.shape
    cw = n // N_GRP
    n_exch = 15

    ops = []
    t = 0
    K = 2
    while K <= N_DEV:
        d = K // 2
        while d >= 1:
            ops.append(("ex", t, K, d))
            t += 1
            d //= 2
        ops.append(("loc", K))
        K *= 2

    def body(x_ref, out_ref, work_ref, staging_ref, send_sems, recv_sems):
        p = lax.axis_index("i")
        gbase = p * m

        partners = [1, 2, 4, 8, 16]
        barrier = pltpu.get_barrier_semaphore()
        for d in partners:
            pl.semaphore_signal(
                barrier,
                inc=1,
                device_id=(jnp.bitwise_xor(p, d),),
                device_id_type=pl.DeviceIdType.MESH,
            )
        pl.semaphore_wait(barrier, len(partners))

        def make_ex(t, d, g):
            return pltpu.make_async_remote_copy(
                src_ref=work_ref.at[g],
                dst_ref=staging_ref.at[t, g],
                send_sem=send_sems.at[t, g],
                recv_sem=recv_sems.at[t, g],
                device_id=(jnp.bitwise_xor(p, d),),
                device_id_type=pl.DeviceIdType.MESH,
            )

        def merge_locals(K, g):
            asc = (p & K) == 0
            for j in (512, 256):
                for b in range(m // (2 * j)):
                    for s in range(j // _RBLK):
                        off = b * 2 * j + s * _RBLK
                        a = work_ref[g, pl.ds(off, _RBLK), :]
                        c = work_ref[g, pl.ds(off + j, _RBLK), :]
                        mn = jnp.minimum(a, c)
                        mx = jnp.maximum(a, c)
                        work_ref[g, pl.ds(off, _RBLK), :] = jnp.where(
                            asc, mn, mx
                        )
                        work_ref[g, pl.ds(off + j, _RBLK), :] = jnp.where(
                            asc, mx, mn
                        )
            for tblk in range(m // _RBLK):
                rows = pl.ds(tblk * _RBLK, _RBLK)
                xv = work_ref[g, rows, :]
                for j in (128, 64, 32, 16, 8):
                    pieces = []
                    for b in range(_RBLK // (2 * j)):
                        a = xv[b * 2 * j : b * 2 * j + j]
                        c = xv[b * 2 * j + j : b * 2 * j + 2 * j]
                        mn = jnp.minimum(a, c)
                        mx = jnp.maximum(a, c)
                        pieces.append(jnp.where(asc, mn, mx))
                        pieces.append(jnp.where(asc, mx, mn))
                    xv = jnp.concatenate(pieces, axis=0)
                for j in (4, 2, 1):
                    gg = _RBLK // (2 * j)
                    y = xv.reshape(gg, 2, j, cw)
                    a = y[:, 0]
                    c = y[:, 1]
                    mn = jnp.minimum(a, c)
                    mx = jnp.maximum(a, c)
                    first = jnp.where(asc, mn, mx)
                    second = jnp.where(asc, mx, mn)
                    xv = jnp.stack([first, second], axis=1).reshape(
                        _RBLK, cw
                    )
                work_ref[g, rows, :] = xv

        def keep(t, K, d, g):
            asc = (p & K) == 0
            lower = (p & d) == 0
            keep_min = asc == lower
            for s in range(m // _RBLK):
                rows = pl.ds(s * _RBLK, _RBLK)
                a = work_ref[g, rows, :]
                b = staging_ref[t, g, rows, :]
                mn = jnp.minimum(a, b)
                mx = jnp.maximum(a, b)
                work_ref[g, rows, :] = jnp.where(keep_min, mn, mx)

        rdmas = {}

        for g in range(N_GRP):
            for s in range(m // _RBLK):
                rows = pl.ds(s * _RBLK, _RBLK)
                work_ref[g, rows, :] = x_ref[rows, pl.ds(g * cw, cw)]
            k = 2
            while k <= m:
                j = k // 2
                while j >= 1:
                    _cmpex_phase(work_ref, g, k, j, m, gbase)
                    j //= 2
                k *= 2
            rdmas[(0, g)] = make_ex(0, ops[0][3], g)
            rdmas[(0, g)].start()

        for idx, op in enumerate(ops):
            for g in range(N_GRP):
                if op[0] == "ex":
                    _, t, K, d = op
                    rdmas[(t, g)].wait()
                    keep(t, K, d, g)
                else:
                    merge_locals(op[1], g)
                if idx + 1 < len(ops) and ops[idx + 1][0] == "ex":
                    t2, d2 = ops[idx + 1][1], ops[idx + 1][3]
                    rdmas[(t2, g)] = make_ex(t2, d2, g)
                    rdmas[(t2, g)].start()

        for g in range(N_GRP):
            for s in range(m // _RBLK):
                rows = pl.ds(s * _RBLK, _RBLK)
                out_ref[rows, pl.ds(g * cw, cw)] = work_ref[g, rows, :]

        def _exit(second_barrier):
            for d in partners:
                pl.semaphore_signal(
                    second_barrier,
                    inc=1,
                    device_id=(jnp.bitwise_xor(p, d),),
                    device_id_type=pl.DeviceIdType.MESH,
                )
            pl.semaphore_wait(second_barrier, len(partners))

        pl.run_scoped(_exit, second_barrier=pltpu.SemaphoreType.REGULAR)

    return pl.pallas_call(
        body,
        out_shape=jax.ShapeDtypeStruct((m, n), x.dtype),
        in_specs=[pl.BlockSpec(memory_space=pltpu.VMEM)],
        out_specs=pl.BlockSpec(memory_space=pltpu.VMEM),
        scratch_shapes=[
            pltpu.VMEM((N_GRP, m, cw), x.dtype),
            pltpu.VMEM((n_exch, N_GRP, m, cw), x.dtype),
            pltpu.SemaphoreType.DMA((n_exch, N_GRP)),
            pltpu.SemaphoreType.DMA((n_exch, N_GRP)),
        ],
        compiler_params=pltpu.CompilerParams(
            collective_id=0,
            vmem_limit_bytes=100 * 1024 * 1024,
        ),
    )(x)
